# Initial kernel scaffold; baseline (speedup 1.0000x reference)
import jax
import jax.numpy as jnp
from jax.experimental import pallas as pl


def kernel(cond_codes, cond_parents, proc_codes, proc_parents, drug_codes, cond_last_parents, proc_last_parents, E_cond, E_cond_parent, E_proc, E_proc_parent, E_drug, W1, b1, W2, K_cond, K_proc, Wi_cond, Wh_cond, bi_cond, bh_cond, Wi_proc, Wh_proc, bi_proc, bh_proc, Wi_drug, Wh_drug, bi_drug, bh_drug, W_fc, b_fc):
    raise NotImplementedError("write your pallas kernel here")


# speedup vs baseline: 4.4713x; 4.4713x over previous
import functools

import jax
import jax.numpy as jnp
from jax import lax
from jax.experimental import pallas as pl
from jax.experimental.pallas import tpu as pltpu
from jax.experimental.pallas import tpu_sc as plsc

B, V, C, P, D = 64, 10, 8, 3, 128
NCODE = B * V * C
NPAR = NCODE * P
NK = B * C * P
N_EMB = 2 * NCODE
N_IDX = N_EMB + 2 * NPAR + NCODE + 2 * NK
CHUNK = 128
N_CHUNKS = N_IDX // CHUNK
NWORK = 32
CPW = N_CHUNKS // NWORK

ROW_PAR = N_EMB
ROW_DRUG = ROW_PAR + 2 * NPAR
ROW_K = ROW_DRUG + NCODE

OFF_EC, OFF_EPC, OFF_EP, OFF_EPP, OFF_ED = 0, 10000, 11000, 19000, 19800
TBL_ROWS = 20400

GRID1 = 8
EBLK = N_EMB // GRID1
DBLK = NCODE // GRID1
VBLK = EBLK // C
VDBLK = DBLK // C


def _dotT(a, b):
    return lax.dot_general(a, b, (((1,), (1,)), ((), ())),
                           preferred_element_type=jnp.float32)


@functools.cache
def _make_sc_gather():
    @functools.partial(
        pl.kernel,
        mesh=plsc.VectorSubcoreMesh(core_axis_name="c", subcore_axis_name="s"),
        out_type=jax.ShapeDtypeStruct((N_IDX, D), jnp.float32),
        scratch_types=[
            pltpu.VMEM((CPW, CHUNK), jnp.int32),
            pltpu.VMEM((CHUNK, D), jnp.float32),
            pltpu.VMEM((CHUNK, D), jnp.float32),
            pltpu.SemaphoreType.DMA,
            pltpu.SemaphoreType.DMA,
        ],
    )
    def _sc_gather(tbl_hbm, idx_hbm, out_hbm, idx_v, rowA, rowB, semA, semB):
        wid = lax.axis_index("s") * 2 + lax.axis_index("c")
        base_chunk = wid * CPW
        pltpu.sync_copy(idx_hbm.at[wid], idx_v)
        bufs = (rowA, rowB)
        sems = (semA, semB)
        pending = pltpu.async_copy(tbl_hbm.at[idx_v.at[0]], bufs[0], sems[0])
        for j in range(CPW):
            nxt = None
            if j + 1 < CPW:
                nxt = pltpu.async_copy(
                    tbl_hbm.at[idx_v.at[j + 1]], bufs[(j + 1) % 2],
                    sems[(j + 1) % 2])
            pending.wait()
            pltpu.sync_copy(bufs[j % 2],
                            out_hbm.at[pl.ds((base_chunk + j) * CHUNK, CHUNK)])
            pending = nxt
    return _sc_gather


def _attn_body(emb_ref, p0_ref, p1_ref, p2_ref, drug_ref,
               W1_ref, b1_ref, W2_ref, v_ref, vd_ref):
    emb = emb_ref[...]
    W1 = W1_ref[...]
    W1a = W1[:, :D]
    W1b = W1[:, D:]
    b1 = b1_ref[...]
    w2 = W2_ref[...]
    Ha = _dotT(emb, W1a) + b1
    hs = jnp.tanh(Ha + _dotT(emb, W1b))
    ss = _dotT(hs, w2)
    ps = (p0_ref[...], p1_ref[...], p2_ref[...])
    scs = []
    for p in ps:
        hj = jnp.tanh(Ha + _dotT(p, W1b))
        scs.append(_dotT(hj, w2))
    m = jnp.maximum(jnp.maximum(scs[0], scs[1]), jnp.maximum(scs[2], ss))
    es = jnp.exp(ss - m)
    num = es * emb
    den = es
    for p, s in zip(ps, scs):
        e = jnp.exp(s - m)
        num = num + e * p
        den = den + e
    ce = num / den
    v_ref[...] = jnp.sum(ce.reshape(VBLK, C, D), axis=1)
    vd_ref[...] = jnp.sum(drug_ref[...].reshape(VDBLK, C, D), axis=1)


_TC1_IN_SPECS = [
    pl.BlockSpec((EBLK, D), lambda i: (i, 0)),
    pl.BlockSpec((EBLK, D), lambda i: (GRID1 + i, 0)),
    pl.BlockSpec((EBLK, D), lambda i: (2 * GRID1 + i, 0)),
    pl.BlockSpec((EBLK, D), lambda i: (3 * GRID1 + i, 0)),
    pl.BlockSpec((DBLK, D), lambda i: (ROW_DRUG // DBLK + i, 0)),
    pl.BlockSpec((D, 2 * D), lambda i: (0, 0)),
    pl.BlockSpec((1, D), lambda i: (0, 0)),
    pl.BlockSpec((1, D), lambda i: (0, 0)),
]
_TC1_OUT_SPECS = [
    pl.BlockSpec((VBLK, D), lambda i: (i, 0)),
    pl.BlockSpec((VDBLK, D), lambda i: (i, 0)),
]
_TC1_OUT_SHAPE = [
    jax.ShapeDtypeStruct((2 * B * V, D), jnp.float32),
    jax.ShapeDtypeStruct((B * V, D), jnp.float32),
]


def _seq_body(va_ref, vd_ref, k_ref,
              Wi_c, Wh_c, bi_c, bh_c, Wi_p, Wh_p, bi_p, bh_p,
              Wi_d, Wh_d, bi_d, bh_d, Kc_ref, Kp_ref, Wfc_ref, bfc_ref,
              out_ref):
    va = va_ref[...]
    vd = vd_ref[...]

    def gru(x, Wi_r, Wh_r, bi_r, bh_r):
        Wi = Wi_r[...]
        Wh = Wh_r[...]
        gi = _dotT(x, Wi) + bi_r[...]
        h = jnp.zeros((B, D), jnp.float32)
        bh = bh_r[...]
        for t in range(V):
            git = gi[t * B:(t + 1) * B]
            gh = _dotT(h, Wh) + bh
            r = jax.nn.sigmoid(git[:, :D] + gh[:, :D])
            z = jax.nn.sigmoid(git[:, D:2 * D] + gh[:, D:2 * D])
            n = jnp.tanh(git[:, 2 * D:] + r * gh[:, 2 * D:])
            h = (1.0 - z) * n + z * h
        return h

    h_c = gru(va[:B * V], Wi_c, Wh_c, bi_c, bh_c)
    h_p = gru(va[B * V:], Wi_p, Wh_p, bi_p, bh_p)
    h_d = gru(vd, Wi_d, Wh_d, bi_d, bh_d)
    tmp = h_c + h_p + h_d

    krows = k_ref[...]

    def knowledge(rows, K_r):
        kp = _dotT(rows, K_r[...])
        kp3 = kp.reshape(B, C * P, D)
        w = jnp.sum(kp3 * tmp.reshape(B, 1, D), axis=2)
        m = jnp.max(w, axis=1, keepdims=True)
        e = jnp.exp(w - m)
        a = e / jnp.sum(e, axis=1, keepdims=True)
        return jnp.sum(a[:, :, None] * kp3, axis=1)

    k_c = knowledge(krows[:NK], Kc_ref)
    k_p = knowledge(krows[NK:], Kp_ref)
    patient = jnp.concatenate([h_c, h_p, h_d, k_c, k_p], axis=1)
    out_ref[...] = _dotT(patient, Wfc_ref[...]) + bfc_ref[...]


_TC2_IN_SPECS = [
    pl.BlockSpec((2 * B * V, D), lambda i: (0, 0)),
    pl.BlockSpec((B * V, D), lambda i: (0, 0)),
    pl.BlockSpec((2 * NK, D), lambda i: (ROW_K // (2 * NK), 0)),
] + [pl.BlockSpec((3 * D, D), lambda i: (0, 0)),
     pl.BlockSpec((3 * D, D), lambda i: (0, 0)),
     pl.BlockSpec((1, 3 * D), lambda i: (0, 0)),
     pl.BlockSpec((1, 3 * D), lambda i: (0, 0)),
     ] * 3 + [
    pl.BlockSpec((D, D), lambda i: (0, 0)),
    pl.BlockSpec((D, D), lambda i: (0, 0)),
    pl.BlockSpec((D, 5 * D), lambda i: (0, 0)),
    pl.BlockSpec((1, D), lambda i: (0, 0)),
]
_TC2_OUT_SPECS = pl.BlockSpec((B, D), lambda i: (0, 0))
_TC2_OUT_SHAPE = jax.ShapeDtypeStruct((B, D), jnp.float32)


def _flat_indices(cond_codes, cond_parents, proc_codes, proc_parents,
                  drug_codes, cond_last_parents, proc_last_parents):
    ce = cond_codes.transpose(1, 0, 2).reshape(-1) + OFF_EC
    pe = proc_codes.transpose(1, 0, 2).reshape(-1) + OFF_EP
    cp = cond_parents.transpose(3, 1, 0, 2).reshape(P, -1) + OFF_EPC
    pp = proc_parents.transpose(3, 1, 0, 2).reshape(P, -1) + OFF_EPP
    par = jnp.concatenate([cp, pp], axis=1).reshape(-1)
    dr = drug_codes.transpose(1, 0, 2).reshape(-1) + OFF_ED
    kc = cond_last_parents.reshape(-1) + OFF_EPC
    kp = proc_last_parents.reshape(-1) + OFF_EPP
    idx = jnp.concatenate([ce, pe, par, dr, kc, kp]).astype(jnp.int32)
    return idx.reshape(NWORK, CPW, CHUNK)


def kernel(cond_codes, cond_parents, proc_codes, proc_parents, drug_codes,
           cond_last_parents, proc_last_parents, E_cond, E_cond_parent,
           E_proc, E_proc_parent, E_drug, W1, b1, W2, K_cond, K_proc,
           Wi_cond, Wh_cond, bi_cond, bh_cond, Wi_proc, Wh_proc, bi_proc,
           bh_proc, Wi_drug, Wh_drug, bi_drug, bh_drug, W_fc, b_fc):
    tbl = jnp.concatenate([E_cond, E_cond_parent, E_proc, E_proc_parent,
                           E_drug], axis=0)
    idx2 = _flat_indices(cond_codes, cond_parents, proc_codes, proc_parents,
                         drug_codes, cond_last_parents, proc_last_parents)
    G = _make_sc_gather()(tbl, idx2)

    va, vd = pl.pallas_call(
        _attn_body,
        grid=(GRID1,),
        in_specs=_TC1_IN_SPECS,
        out_specs=_TC1_OUT_SPECS,
        out_shape=_TC1_OUT_SHAPE,
    )(G, G, G, G, G, W1, b1.reshape(1, D), W2)

    out = pl.pallas_call(
        _seq_body,
        grid=(1,),
        in_specs=_TC2_IN_SPECS,
        out_specs=_TC2_OUT_SPECS,
        out_shape=_TC2_OUT_SHAPE,
    )(va, vd, G,
      Wi_cond, Wh_cond, bi_cond.reshape(1, 3 * D), bh_cond.reshape(1, 3 * D),
      Wi_proc, Wh_proc, bi_proc.reshape(1, 3 * D), bh_proc.reshape(1, 3 * D),
      Wi_drug, Wh_drug, bi_drug.reshape(1, 3 * D), bh_drug.reshape(1, 3 * D),
      K_cond, K_proc, W_fc, b_fc.reshape(1, D))
    return out


# speedup vs baseline: 4.4800x; 1.0019x over previous
import functools

import jax
import jax.numpy as jnp
from jax import lax
from jax.experimental import pallas as pl
from jax.experimental.pallas import tpu as pltpu
from jax.experimental.pallas import tpu_sc as plsc

B, V, C, P, D = 64, 10, 8, 3, 128
NCODE = B * V * C
NPAR = NCODE * P
NK = B * C * P
N_EMB = 2 * NCODE
N_IDX = N_EMB + 2 * NPAR + NCODE + 2 * NK
CHUNK = 128
N_CHUNKS = N_IDX // CHUNK
NWORK = 32
CPW = N_CHUNKS // NWORK

ROW_PAR = N_EMB
ROW_DRUG = ROW_PAR + 2 * NPAR
ROW_K = ROW_DRUG + NCODE

OFF_EC, OFF_EPC, OFF_EP, OFF_EPP, OFF_ED = 0, 10000, 11000, 19000, 19800
TBL_ROWS = 20400

GRID1 = 8
EBLK = N_EMB // GRID1
DBLK = NCODE // GRID1
VBLK = EBLK // C
VDBLK = DBLK // C


def _dotT(a, b):
    return lax.dot_general(a, b, (((1,), (1,)), ((), ())),
                           preferred_element_type=jnp.float32)


@functools.cache
def _make_sc_gather():
    @functools.partial(
        pl.kernel,
        mesh=plsc.VectorSubcoreMesh(core_axis_name="c", subcore_axis_name="s"),
        out_type=jax.ShapeDtypeStruct((N_IDX, D), jnp.float32),
        compiler_params=pltpu.CompilerParams(use_tc_tiling_on_sc=True),
        scratch_types=[
            pltpu.VMEM((CPW, CHUNK), jnp.int32),
            pltpu.VMEM((CHUNK, D), jnp.float32),
            pltpu.VMEM((CHUNK, D), jnp.float32),
            pltpu.SemaphoreType.DMA,
            pltpu.SemaphoreType.DMA,
        ],
    )
    def _sc_gather(tbl_hbm, idx_hbm, out_hbm, idx_v, rowA, rowB, semA, semB):
        wid = lax.axis_index("s") * 2 + lax.axis_index("c")
        base_chunk = wid * CPW
        pltpu.sync_copy(idx_hbm.at[wid], idx_v)
        bufs = (rowA, rowB)
        sems = (semA, semB)
        pending = pltpu.async_copy(tbl_hbm.at[idx_v.at[0]], bufs[0], sems[0])
        for j in range(CPW):
            nxt = None
            if j + 1 < CPW:
                nxt = pltpu.async_copy(
                    tbl_hbm.at[idx_v.at[j + 1]], bufs[(j + 1) % 2],
                    sems[(j + 1) % 2])
            pending.wait()
            pltpu.sync_copy(bufs[j % 2],
                            out_hbm.at[pl.ds((base_chunk + j) * CHUNK, CHUNK)])
            pending = nxt
    return _sc_gather


def _attn_body(emb_ref, p0_ref, p1_ref, p2_ref, drug_ref,
               W1_ref, b1_ref, W2_ref, v_ref, vd_ref):
    emb = emb_ref[...]
    W1 = W1_ref[...]
    W1a = W1[:, :D]
    W1b = W1[:, D:]
    b1 = b1_ref[...]
    w2 = W2_ref[...]
    Ha = _dotT(emb, W1a) + b1
    hs = jnp.tanh(Ha + _dotT(emb, W1b))
    ss = _dotT(hs, w2)
    ps = (p0_ref[...], p1_ref[...], p2_ref[...])
    scs = []
    for p in ps:
        hj = jnp.tanh(Ha + _dotT(p, W1b))
        scs.append(_dotT(hj, w2))
    m = jnp.maximum(jnp.maximum(scs[0], scs[1]), jnp.maximum(scs[2], ss))
    es = jnp.exp(ss - m)
    num = es * emb
    den = es
    for p, s in zip(ps, scs):
        e = jnp.exp(s - m)
        num = num + e * p
        den = den + e
    ce = num / den
    v_ref[...] = jnp.sum(ce.reshape(VBLK, C, D), axis=1)
    vd_ref[...] = jnp.sum(drug_ref[...].reshape(VDBLK, C, D), axis=1)


_TC1_IN_SPECS = [
    pl.BlockSpec((EBLK, D), lambda i: (i, 0)),
    pl.BlockSpec((EBLK, D), lambda i: (GRID1 + i, 0)),
    pl.BlockSpec((EBLK, D), lambda i: (2 * GRID1 + i, 0)),
    pl.BlockSpec((EBLK, D), lambda i: (3 * GRID1 + i, 0)),
    pl.BlockSpec((DBLK, D), lambda i: (ROW_DRUG // DBLK + i, 0)),
    pl.BlockSpec((D, 2 * D), lambda i: (0, 0)),
    pl.BlockSpec((1, D), lambda i: (0, 0)),
    pl.BlockSpec((1, D), lambda i: (0, 0)),
]
_TC1_OUT_SPECS = [
    pl.BlockSpec((VBLK, D), lambda i: (i, 0)),
    pl.BlockSpec((VDBLK, D), lambda i: (i, 0)),
]
_TC1_OUT_SHAPE = [
    jax.ShapeDtypeStruct((2 * B * V, D), jnp.float32),
    jax.ShapeDtypeStruct((B * V, D), jnp.float32),
]


def _seq_body(va_ref, vd_ref, k_ref,
              Wi_c, Wh_c, bi_c, bh_c, Wi_p, Wh_p, bi_p, bh_p,
              Wi_d, Wh_d, bi_d, bh_d, Kc_ref, Kp_ref, Wfc_ref, bfc_ref,
              out_ref):
    va = va_ref[...]
    vd = vd_ref[...]

    def gru(x, Wi_r, Wh_r, bi_r, bh_r):
        Wi = Wi_r[...]
        Wh = Wh_r[...]
        gi = _dotT(x, Wi) + bi_r[...]
        h = jnp.zeros((B, D), jnp.float32)
        bh = bh_r[...]
        for t in range(V):
            git = gi[t * B:(t + 1) * B]
            gh = _dotT(h, Wh) + bh
            r = jax.nn.sigmoid(git[:, :D] + gh[:, :D])
            z = jax.nn.sigmoid(git[:, D:2 * D] + gh[:, D:2 * D])
            n = jnp.tanh(git[:, 2 * D:] + r * gh[:, 2 * D:])
            h = (1.0 - z) * n + z * h
        return h

    h_c = gru(va[:B * V], Wi_c, Wh_c, bi_c, bh_c)
    h_p = gru(va[B * V:], Wi_p, Wh_p, bi_p, bh_p)
    h_d = gru(vd, Wi_d, Wh_d, bi_d, bh_d)
    tmp = h_c + h_p + h_d

    krows = k_ref[...]

    def knowledge(rows, K_r):
        kp = _dotT(rows, K_r[...])
        kp3 = kp.reshape(B, C * P, D)
        w = jnp.sum(kp3 * tmp.reshape(B, 1, D), axis=2)
        m = jnp.max(w, axis=1, keepdims=True)
        e = jnp.exp(w - m)
        a = e / jnp.sum(e, axis=1, keepdims=True)
        return jnp.sum(a[:, :, None] * kp3, axis=1)

    k_c = knowledge(krows[:NK], Kc_ref)
    k_p = knowledge(krows[NK:], Kp_ref)
    patient = jnp.concatenate([h_c, h_p, h_d, k_c, k_p], axis=1)
    out_ref[...] = _dotT(patient, Wfc_ref[...]) + bfc_ref[...]


_TC2_IN_SPECS = [
    pl.BlockSpec((2 * B * V, D), lambda i: (0, 0)),
    pl.BlockSpec((B * V, D), lambda i: (0, 0)),
    pl.BlockSpec((2 * NK, D), lambda i: (ROW_K // (2 * NK), 0)),
] + [pl.BlockSpec((3 * D, D), lambda i: (0, 0)),
     pl.BlockSpec((3 * D, D), lambda i: (0, 0)),
     pl.BlockSpec((1, 3 * D), lambda i: (0, 0)),
     pl.BlockSpec((1, 3 * D), lambda i: (0, 0)),
     ] * 3 + [
    pl.BlockSpec((D, D), lambda i: (0, 0)),
    pl.BlockSpec((D, D), lambda i: (0, 0)),
    pl.BlockSpec((D, 5 * D), lambda i: (0, 0)),
    pl.BlockSpec((1, D), lambda i: (0, 0)),
]
_TC2_OUT_SPECS = pl.BlockSpec((B, D), lambda i: (0, 0))
_TC2_OUT_SHAPE = jax.ShapeDtypeStruct((B, D), jnp.float32)


def _flat_indices(cond_codes, cond_parents, proc_codes, proc_parents,
                  drug_codes, cond_last_parents, proc_last_parents):
    ce = cond_codes.transpose(1, 0, 2).reshape(-1) + OFF_EC
    pe = proc_codes.transpose(1, 0, 2).reshape(-1) + OFF_EP
    cp = cond_parents.transpose(3, 1, 0, 2).reshape(P, -1) + OFF_EPC
    pp = proc_parents.transpose(3, 1, 0, 2).reshape(P, -1) + OFF_EPP
    par = jnp.concatenate([cp, pp], axis=1).reshape(-1)
    dr = drug_codes.transpose(1, 0, 2).reshape(-1) + OFF_ED
    kc = cond_last_parents.reshape(-1) + OFF_EPC
    kp = proc_last_parents.reshape(-1) + OFF_EPP
    idx = jnp.concatenate([ce, pe, par, dr, kc, kp]).astype(jnp.int32)
    return idx.reshape(NWORK, CPW, CHUNK)


def kernel(cond_codes, cond_parents, proc_codes, proc_parents, drug_codes,
           cond_last_parents, proc_last_parents, E_cond, E_cond_parent,
           E_proc, E_proc_parent, E_drug, W1, b1, W2, K_cond, K_proc,
           Wi_cond, Wh_cond, bi_cond, bh_cond, Wi_proc, Wh_proc, bi_proc,
           bh_proc, Wi_drug, Wh_drug, bi_drug, bh_drug, W_fc, b_fc):
    tbl = jnp.concatenate([E_cond, E_cond_parent, E_proc, E_proc_parent,
                           E_drug], axis=0)
    idx2 = _flat_indices(cond_codes, cond_parents, proc_codes, proc_parents,
                         drug_codes, cond_last_parents, proc_last_parents)
    G = _make_sc_gather()(tbl, idx2)

    va, vd = pl.pallas_call(
        _attn_body,
        grid=(GRID1,),
        in_specs=_TC1_IN_SPECS,
        out_specs=_TC1_OUT_SPECS,
        out_shape=_TC1_OUT_SHAPE,
    )(G, G, G, G, G, W1, b1.reshape(1, D), W2)

    out = pl.pallas_call(
        _seq_body,
        grid=(1,),
        in_specs=_TC2_IN_SPECS,
        out_specs=_TC2_OUT_SPECS,
        out_shape=_TC2_OUT_SHAPE,
    )(va, vd, G,
      Wi_cond, Wh_cond, bi_cond.reshape(1, 3 * D), bh_cond.reshape(1, 3 * D),
      Wi_proc, Wh_proc, bi_proc.reshape(1, 3 * D), bh_proc.reshape(1, 3 * D),
      Wi_drug, Wh_drug, bi_drug.reshape(1, 3 * D), bh_drug.reshape(1, 3 * D),
      K_cond, K_proc, W_fc, b_fc.reshape(1, D))
    return out
